# Initial kernel scaffold; baseline (speedup 1.0000x reference)
import jax
import jax.numpy as jnp
from jax.experimental import pallas as pl


def kernel(feats, edge_index, W_enc, b_enc, W_gin, b_gin, gamma, beta, W_cls, b_cls):
    raise NotImplementedError("write your pallas kernel here")


# speedup vs baseline: 4.8632x; 4.8632x over previous
import functools

import jax
import jax.numpy as jnp
from jax import lax
from jax.experimental import pallas as pl
from jax.experimental.pallas import tpu as pltpu
from jax.experimental.pallas import tpu_sc as plsc

N = 10000
E = 320000
RANK = 128
ORDER = 3
BN_EPS = 1e-5
BS = 100
FLOW = 100

NC = 2
NS = 16
NW = NC * NS
CHUNK = 128
EPW_CHUNKS = 80
E_PAD = NW * EPW_CHUNKS * CHUNK
N_PAD = 10240
ROWS_PER_SUB = N_PAD // NS
INIT_CHUNKS = ROWS_PER_SUB // CHUNK


def _sc_body(x_hbm, src_hbm, dst_hbm, z_hbm, out_hbm, idx_s, idx_d, rows,
             acc, sem):
    c = lax.axis_index("c")
    s = lax.axis_index("s")
    wid = s * NC + c

    pltpu.sync_copy(src_hbm.at[pl.ds(wid * EPW_CHUNKS, EPW_CHUNKS)], idx_s)
    pltpu.sync_copy(dst_hbm.at[pl.ds(wid * EPW_CHUNKS, EPW_CHUNKS)], idx_d)

    pltpu.sync_copy(z_hbm, rows)

    def init_step(k, carry):
        r0 = s * ROWS_PER_SUB + k * CHUNK
        pltpu.sync_copy(rows, acc.at[pl.ds(r0, CHUNK)])
        return carry

    lax.fori_loop(0, INIT_CHUNKS, init_step, 0)
    plsc.subcore_barrier()

    def edge_step(j, carry):
        pltpu.async_copy(x_hbm.at[idx_s.at[j]], rows, sem).wait()
        pltpu.sync_copy(rows, acc.at[idx_d.at[j]], add=True)
        return carry

    lax.fori_loop(0, EPW_CHUNKS, edge_step, 0)
    plsc.subcore_barrier()

    def out_step(k, carry):
        r0 = s * ROWS_PER_SUB + k * CHUNK
        pltpu.sync_copy(acc.at[pl.ds(r0, CHUNK)], rows)
        pltpu.sync_copy(rows, out_hbm.at[c, pl.ds(r0, CHUNK)])
        return carry

    lax.fori_loop(0, INIT_CHUNKS, out_step, 0)


@functools.cache
def _sc_gather_scatter():
    return pl.kernel(
        _sc_body,
        out_type=jax.ShapeDtypeStruct((NC, N_PAD, RANK), jnp.float32),
        mesh=plsc.VectorSubcoreMesh(core_axis_name="c", subcore_axis_name="s",
                                    num_cores=NC, num_subcores=NS),
        scratch_types=[
            pltpu.VMEM((EPW_CHUNKS, CHUNK), jnp.int32),
            pltpu.VMEM((EPW_CHUNKS, CHUNK), jnp.int32),
            pltpu.VMEM((CHUNK, RANK), jnp.float32),
            pltpu.VMEM_SHARED((N_PAD, RANK), jnp.float32),
            pltpu.SemaphoreType.DMA,
        ],
    )


def _enc_body(feats_ref, wenc_ref, benc_ref, out_ref):
    out_ref[:, :] = feats_ref[:, :] * wenc_ref[0, :] + benc_ref[:]


def _encoder(feats_pad, W_enc, b_enc):
    return pl.pallas_call(
        _enc_body,
        out_shape=jax.ShapeDtypeStruct((N_PAD, RANK), jnp.float32),
    )(feats_pad, W_enc, b_enc)


def _mlp_body(x_ref, p_ref, W_ref, b_ref, g_ref, be_ref, wc_ref, bc_ref,
              xout_ref, yp_ref):
    h = x_ref[:, :] + p_ref[0] + p_ref[1]
    mask = lax.broadcasted_iota(jnp.int32, (N_PAD, 1), 0) < N
    inv_n = 1.0 / N
    for j in range(3):
        h = jnp.dot(h, W_ref[j], preferred_element_type=jnp.float32) + b_ref[j]
        h = jnp.maximum(h, 0.0)
        hm = jnp.where(mask, h, 0.0)
        mean = jnp.sum(hm, axis=0, keepdims=True) * inv_n
        d = h - mean
        dm = jnp.where(mask, d, 0.0)
        var = jnp.sum(dm * dm, axis=0, keepdims=True) * inv_n
        h = d * lax.rsqrt(var + BN_EPS) * g_ref[j] + be_ref[j]
    xout_ref[:, :] = h
    gsel = (lax.broadcasted_iota(jnp.int32, (BS, N_PAD), 1) // FLOW
            == lax.broadcasted_iota(jnp.int32, (BS, N_PAD), 0))
    r = jnp.dot(gsel.astype(jnp.float32), h, preferred_element_type=jnp.float32)
    yp_ref[:, :] = (jnp.dot(r, wc_ref[:, :], preferred_element_type=jnp.float32)
                    + bc_ref[:])


def _mlp_layer(x, p, W, b, g, be, wc, bc):
    return pl.pallas_call(
        _mlp_body,
        out_shape=(
            jax.ShapeDtypeStruct((N_PAD, RANK), jnp.float32),
            jax.ShapeDtypeStruct((BS, wc.shape[1]), jnp.float32),
        ),
    )(x, p, W, b, g, be, wc, bc)


def kernel(feats, edge_index, W_enc, b_enc, W_gin, b_gin, gamma, beta,
           W_cls, b_cls):
    src = edge_index[0].astype(jnp.int32)
    dst = edge_index[1].astype(jnp.int32)
    perm = jnp.argsort(dst, stable=True)
    src = src[perm]
    dst = dst[perm]
    pad_e = E_PAD - E
    pad_idx = N + (jnp.arange(pad_e, dtype=jnp.int32) % (N_PAD - N))
    src_p = jnp.concatenate([src, pad_idx]).reshape(NW * EPW_CHUNKS, CHUNK)
    dst_p = jnp.concatenate([dst, pad_idx]).reshape(NW * EPW_CHUNKS, CHUNK)
    feats_p = jnp.concatenate(
        [feats.reshape(N, 1), jnp.zeros((N_PAD - N, 1), jnp.float32)], axis=0)
    zrows = jnp.zeros((CHUNK, RANK), jnp.float32)

    x = _encoder(feats_p, W_enc, b_enc)
    zeros_bc = jnp.zeros_like(b_cls)
    y = None
    for i in range(ORDER):
        p = _sc_gather_scatter()(x, src_p, dst_p, zrows)
        bc_i = b_cls if i == 0 else zeros_bc
        x, yp = _mlp_layer(x, p, W_gin[i], b_gin[i], gamma[i], beta[i],
                           W_cls[i * RANK:(i + 1) * RANK], bc_i)
        y = yp if y is None else y + yp
    return y
